# Initial kernel scaffold; baseline (speedup 1.0000x reference)
import jax
import jax.numpy as jnp
from jax.experimental import pallas as pl


def kernel(x, edge_index, W1, b1, g1, be1, W2, b2, g2, be2, W3, b3, g3, be3):
    raise NotImplementedError("write your pallas kernel here")


# speedup vs baseline: 16.2119x; 16.2119x over previous
import functools

import jax
import jax.numpy as jnp
from jax import lax
from jax.experimental import pallas as pl
from jax.experimental.pallas import tpu as pltpu
from jax.experimental.pallas import tpu_sc as plsc

N = 10000
E = 320000
NC = 2
NS = 16
TILES = NC * NS
CH = 128
CPT = -(-E // (TILES * CH))
EPAD = TILES * CPT * CH
NPAD = 10112
RPT = NPAD // NS
EPS = 1e-5

_MESH = plsc.VectorSubcoreMesh(core_axis_name="c", subcore_axis_name="s")


def _make_agg(width):
    @functools.partial(
        pl.kernel,
        out_type=jax.ShapeDtypeStruct((NC, NPAD, width), jnp.float32),
        mesh=_MESH,
        scratch_types=[
            pltpu.VMEM((CPT, CH), jnp.int32),
            pltpu.VMEM((CPT, CH), jnp.int32),
            pltpu.VMEM((CH, width), jnp.float32),
            pltpu.VMEM_SHARED((NPAD, width), jnp.float32),
            pltpu.SemaphoreType.DMA,
            pltpu.SemaphoreType.DMA,
        ],
        compiler_params=pltpu.CompilerParams(use_tc_tiling_on_sc=False),
    )
    def agg(table, src, dst, zeros, out, idx_s, idx_d, buf, acc, gsem, ssem):
        c = lax.axis_index("c")
        s = lax.axis_index("s")
        wid = c * NS + s
        r0 = s * RPT
        pltpu.sync_copy(zeros.at[pl.ds(r0, RPT)], acc.at[pl.ds(r0, RPT)])
        pltpu.sync_copy(src.at[wid], idx_s)
        pltpu.sync_copy(dst.at[wid], idx_d)
        plsc.subcore_barrier()

        def chunk(j, carry):
            pltpu.async_copy(table.at[idx_s.at[j]], buf, gsem).wait()
            pltpu.async_copy(buf, acc.at[idx_d.at[j]], ssem, add=True).wait()
            return carry

        lax.fori_loop(0, CPT, chunk, 0)
        plsc.subcore_barrier()
        pltpu.sync_copy(acc.at[pl.ds(r0, RPT)], out.at[c, pl.ds(r0, RPT)])

    return agg


_agg64 = _make_agg(64)
_agg16 = _make_agg(16)


def _first_body(x, w, p0, p1, h_out, dinv_out):
    deg = p0[:, 0:1] + p1[:, 0:1] + 1.0
    dinv = lax.rsqrt(deg)
    dinv_out[...] = dinv
    h = jnp.dot(x[...], w[...], preferred_element_type=jnp.float32)
    h_out[0:N, :] = h * dinv[0:N]
    h_out[N:NPAD, :] = jnp.zeros((NPAD - N, h.shape[1]), jnp.float32)


_first = pl.pallas_call(
    _first_body,
    out_shape=(
        jax.ShapeDtypeStruct((NPAD, 64), jnp.float32),
        jax.ShapeDtypeStruct((NPAD, 1), jnp.float32),
    ),
)


def _mid_body(p0, p1, hp, dinv, b, g, be, w, out):
    t = dinv[0:N] * (p0[0:N] + p1[0:N] + hp[0:N]) + b[...]
    mu = jnp.mean(t, axis=0, keepdims=True)
    var = jnp.mean((t - mu) ** 2, axis=0, keepdims=True)
    t = g[...] * (t - mu) * lax.rsqrt(var + EPS) + be[...]
    t = jnp.maximum(t, 0.0)
    h = jnp.dot(t, w[...], preferred_element_type=jnp.float32)
    out[0:N, :] = h * dinv[0:N]
    out[N:NPAD, :] = jnp.zeros((NPAD - N, h.shape[1]), jnp.float32)


def _make_mid(wout):
    return pl.pallas_call(
        _mid_body,
        out_shape=jax.ShapeDtypeStruct((NPAD, wout), jnp.float32),
    )


_mid64 = _make_mid(64)
_mid16 = _make_mid(16)


def _final_body(p0, p1, hp, dinv, b, g, be, out):
    t = dinv[0:N] * (p0[0:N] + p1[0:N] + hp[0:N])
    t = t[:, 0:2] + b[...]
    mu = jnp.mean(t, axis=0, keepdims=True)
    var = jnp.mean((t - mu) ** 2, axis=0, keepdims=True)
    out[...] = g[...] * (t - mu) * lax.rsqrt(var + EPS) + be[...]


_final = pl.pallas_call(
    _final_body,
    out_shape=jax.ShapeDtypeStruct((N, 2), jnp.float32),
)


def kernel(x, edge_index, W1, b1, g1, be1, W2, b2, g2, be2, W3, b3, g3, be3):
    src = edge_index[0].astype(jnp.int32)
    dst = edge_index[1].astype(jnp.int32)
    pad = jnp.full((EPAD - E,), N, jnp.int32)
    srcp = jnp.concatenate([src, pad]).reshape(TILES, CPT, CH)
    dstp = jnp.concatenate([dst, pad]).reshape(TILES, CPT, CH)

    ones16 = jnp.concatenate(
        [jnp.ones((N, 16), jnp.float32), jnp.zeros((NPAD - N, 16), jnp.float32)]
    )
    z16 = jnp.zeros((NPAD, 16), jnp.float32)
    z64 = jnp.zeros((NPAD, 64), jnp.float32)

    pdeg = _agg16(ones16, srcp, dstp, z16)
    h1p, dinv = _first(x, W1, pdeg[0], pdeg[1])

    p1 = _agg64(h1p, srcp, dstp, z64)
    h2p = _mid64(p1[0], p1[1], h1p, dinv,
                 b1.reshape(1, -1), g1.reshape(1, -1), be1.reshape(1, -1), W2)

    p2 = _agg64(h2p, srcp, dstp, z64)
    W3p = jnp.pad(W3, ((0, 0), (0, 16 - W3.shape[1])))
    h3p = _mid16(p2[0], p2[1], h2p, dinv,
                 b2.reshape(1, -1), g2.reshape(1, -1), be2.reshape(1, -1), W3p)

    p3 = _agg16(h3p, srcp, dstp, z16)
    out = _final(p3[0], p3[1], h3p, dinv,
                 b3.reshape(1, -1), g3.reshape(1, -1), be3.reshape(1, -1))
    return out


# speedup vs baseline: 29.3622x; 1.8112x over previous
import functools

import jax
import jax.numpy as jnp
from jax import lax
from jax.experimental import pallas as pl
from jax.experimental.pallas import tpu as pltpu
from jax.experimental.pallas import tpu_sc as plsc

N = 10000
E = 320000
NC = 2
NS = 16
TILES = NC * NS
CH = 128
CPT = 80
CPTP = CPT + 2
NGROUPS = CPT // 2
EPAD = TILES * CPTP * CH
NPAD = 10112
RPT = NPAD // NS
EPS = 1e-5

_MESH = plsc.VectorSubcoreMesh(core_axis_name="c", subcore_axis_name="s")


def _make_agg(width):
    @functools.partial(
        pl.kernel,
        out_type=jax.ShapeDtypeStruct((NC, NPAD, width), jnp.float32),
        mesh=_MESH,
        scratch_types=[
            pltpu.VMEM((CPTP, CH), jnp.int32),
            pltpu.VMEM((CPTP, CH), jnp.int32),
            pltpu.VMEM((2, CH, width), jnp.float32),
            pltpu.VMEM_SHARED((NPAD, width), jnp.float32),
            pltpu.SemaphoreType.DMA,
            pltpu.SemaphoreType.DMA,
        ],
        compiler_params=pltpu.CompilerParams(use_tc_tiling_on_sc=False),
    )
    def agg(table, src, dst, zeros, out, idx_s, idx_d, buf, acc, gsem, ssem):
        c = lax.axis_index("c")
        s = lax.axis_index("s")
        wid = c * NS + s
        r0 = s * RPT
        pltpu.sync_copy(zeros.at[pl.ds(r0, RPT)], acc.at[pl.ds(r0, RPT)])
        pltpu.sync_copy(src.at[wid], idx_s)
        pltpu.sync_copy(dst.at[wid], idx_d)
        plsc.subcore_barrier()

        pltpu.async_copy(table.at[idx_s.at[0]], buf.at[0], gsem)
        pltpu.async_copy(table.at[idx_s.at[1]], buf.at[1], gsem)

        def group(g, carry):
            for b in range(2):
                j = 2 * g + b
                pltpu.make_async_copy(table.at[idx_s.at[j]], buf.at[b], gsem).wait()
                sd = pltpu.async_copy(buf.at[b], acc.at[idx_d.at[j]], ssem, add=True)
                sd.wait()
                pltpu.async_copy(table.at[idx_s.at[j + 2]], buf.at[b], gsem)
            return carry

        lax.fori_loop(0, NGROUPS, group, 0)
        pltpu.make_async_copy(table.at[idx_s.at[CPT]], buf.at[0], gsem).wait()
        pltpu.make_async_copy(table.at[idx_s.at[CPT + 1]], buf.at[1], gsem).wait()
        plsc.subcore_barrier()
        pltpu.sync_copy(acc.at[pl.ds(r0, RPT)], out.at[c, pl.ds(r0, RPT)])

    return agg


_agg64 = _make_agg(64)
_agg16 = _make_agg(16)


def _first_body(x, w, p0, p1, h_out, dinv_out):
    deg = p0[:, 0:1] + p1[:, 0:1] + 1.0
    dinv = lax.rsqrt(deg)
    dinv_out[...] = dinv
    h = jnp.dot(x[...], w[...], preferred_element_type=jnp.float32)
    h_out[0:N, :] = h * dinv[0:N]
    h_out[N:NPAD, :] = jnp.zeros((NPAD - N, h.shape[1]), jnp.float32)


_first = pl.pallas_call(
    _first_body,
    out_shape=(
        jax.ShapeDtypeStruct((NPAD, 64), jnp.float32),
        jax.ShapeDtypeStruct((NPAD, 1), jnp.float32),
    ),
)


def _mid_body(p0, p1, hp, dinv, b, g, be, w, out):
    t = dinv[0:N] * (p0[0:N] + p1[0:N] + hp[0:N]) + b[...]
    mu = jnp.mean(t, axis=0, keepdims=True)
    var = jnp.mean((t - mu) ** 2, axis=0, keepdims=True)
    t = g[...] * (t - mu) * lax.rsqrt(var + EPS) + be[...]
    t = jnp.maximum(t, 0.0)
    h = jnp.dot(t, w[...], preferred_element_type=jnp.float32)
    out[0:N, :] = h * dinv[0:N]
    out[N:NPAD, :] = jnp.zeros((NPAD - N, h.shape[1]), jnp.float32)


def _make_mid(wout):
    return pl.pallas_call(
        _mid_body,
        out_shape=jax.ShapeDtypeStruct((NPAD, wout), jnp.float32),
    )


_mid64 = _make_mid(64)
_mid16 = _make_mid(16)


def _final_body(p0, p1, hp, dinv, b, g, be, out):
    t = dinv[0:N] * (p0[0:N] + p1[0:N] + hp[0:N])
    t = t[:, 0:2] + b[...]
    mu = jnp.mean(t, axis=0, keepdims=True)
    var = jnp.mean((t - mu) ** 2, axis=0, keepdims=True)
    out[...] = g[...] * (t - mu) * lax.rsqrt(var + EPS) + be[...]


_final = pl.pallas_call(
    _final_body,
    out_shape=jax.ShapeDtypeStruct((N, 2), jnp.float32),
)


def kernel(x, edge_index, W1, b1, g1, be1, W2, b2, g2, be2, W3, b3, g3, be3):
    src = edge_index[0].astype(jnp.int32)
    dst = edge_index[1].astype(jnp.int32)
    def _lay(e):
        fpad = N + jnp.arange(TILES * CPT * CH - E, dtype=jnp.int32) % (NPAD - N)
        main = jnp.concatenate([e, fpad]).reshape(TILES, CPT, CH)
        tail = (N + jnp.arange(TILES * 2 * CH, dtype=jnp.int32)
                % (NPAD - N)).reshape(TILES, 2, CH)
        return jnp.concatenate([main, tail], axis=1)

    srcp = _lay(src)
    dstp = _lay(dst)

    ones16 = jnp.concatenate(
        [jnp.ones((N, 16), jnp.float32), jnp.zeros((NPAD - N, 16), jnp.float32)]
    )
    z16 = jnp.zeros((NPAD, 16), jnp.float32)
    z64 = jnp.zeros((NPAD, 64), jnp.float32)

    pdeg = _agg16(ones16, srcp, dstp, z16)
    h1p, dinv = _first(x, W1, pdeg[0], pdeg[1])

    p1 = _agg64(h1p, srcp, dstp, z64)
    h2p = _mid64(p1[0], p1[1], h1p, dinv,
                 b1.reshape(1, -1), g1.reshape(1, -1), be1.reshape(1, -1), W2)

    p2 = _agg64(h2p, srcp, dstp, z64)
    W3p = jnp.pad(W3, ((0, 0), (0, 16 - W3.shape[1])))
    h3p = _mid16(p2[0], p2[1], h2p, dinv,
                 b2.reshape(1, -1), g2.reshape(1, -1), be2.reshape(1, -1), W3p)

    p3 = _agg16(h3p, srcp, dstp, z16)
    out = _final(p3[0], p3[1], h3p, dinv,
                 b3.reshape(1, -1), g3.reshape(1, -1), be3.reshape(1, -1))
    return out


# speedup vs baseline: 35.7531x; 1.2177x over previous
import functools

import jax
import jax.numpy as jnp
from jax import lax
from jax.experimental import pallas as pl
from jax.experimental.pallas import tpu as pltpu
from jax.experimental.pallas import tpu_sc as plsc

N = 10000
E = 320000
NC = 2
NS = 16
TILES = NC * NS
CH = 128
CPT = 80
CPTP = CPT + 2
NPAD = 10112
RPT = NPAD // NS
EPS = 1e-5

_MESH = plsc.VectorSubcoreMesh(core_axis_name="c", subcore_axis_name="s")
_SC_PARAMS = pltpu.CompilerParams(use_tc_tiling_on_sc=False)


def _make_agg(width):
    @functools.partial(
        pl.kernel,
        out_type=jax.ShapeDtypeStruct((NC, NPAD, width), jnp.float32),
        mesh=_MESH,
        scratch_types=[
            pltpu.VMEM((CPTP, CH), jnp.int32),
            pltpu.VMEM((CPTP, CH), jnp.int32),
            pltpu.VMEM((4, CH, width), jnp.float32),
            pltpu.VMEM_SHARED((NPAD, width), jnp.float32),
            pltpu.SemaphoreType.DMA,
            pltpu.SemaphoreType.DMA,
        ],
        compiler_params=_SC_PARAMS,
    )
    def agg(table, src, dst, zeros, out, idx_s, idx_d, buf, acc, gsem, ssem):
        c = lax.axis_index("c")
        s = lax.axis_index("s")
        wid = c * NS + s
        r0 = s * RPT
        pltpu.sync_copy(zeros, acc.at[pl.ds(r0, RPT)])
        pltpu.sync_copy(src.at[wid], idx_s)
        pltpu.sync_copy(dst.at[wid], idx_d)
        plsc.subcore_barrier()

        pltpu.async_copy(table.at[idx_s.at[0]], buf.at[0], gsem)
        pltpu.async_copy(table.at[idx_s.at[1]], buf.at[1], gsem)
        for j in (0, 1):
            pltpu.make_async_copy(table.at[idx_s.at[j]], buf.at[j], gsem).wait()
            pltpu.async_copy(buf.at[j], acc.at[idx_d.at[j]], ssem, add=True)
            pltpu.async_copy(table.at[idx_s.at[j + 2]], buf.at[j + 2], gsem)

        def slot(j, carry):
            b = jnp.bitwise_and(j, 3)
            bm2 = jnp.bitwise_and(j + 2, 3)
            pltpu.make_async_copy(table.at[idx_s.at[j]], buf.at[b], gsem).wait()
            pltpu.async_copy(buf.at[b], acc.at[idx_d.at[j]], ssem, add=True)
            pltpu.make_async_copy(buf.at[bm2], acc.at[idx_d.at[j - 2]], ssem).wait()
            pltpu.async_copy(table.at[idx_s.at[j + 2]], buf.at[bm2], gsem)
            return carry

        lax.fori_loop(2, CPT, slot, 0)
        pltpu.make_async_copy(buf.at[2], acc.at[idx_d.at[CPT - 2]], ssem).wait()
        pltpu.make_async_copy(buf.at[3], acc.at[idx_d.at[CPT - 1]], ssem).wait()
        pltpu.make_async_copy(table.at[idx_s.at[CPT]], buf.at[0], gsem).wait()
        pltpu.make_async_copy(table.at[idx_s.at[CPT + 1]], buf.at[1], gsem).wait()
        plsc.subcore_barrier()
        pltpu.sync_copy(acc.at[pl.ds(r0, RPT)], out.at[c, pl.ds(r0, RPT)])

    return agg


_agg64 = _make_agg(64)
_agg16 = _make_agg(16)


@functools.partial(
    pl.kernel,
    out_type=jax.ShapeDtypeStruct((NC, NPAD, 16), jnp.float32),
    mesh=_MESH,
    scratch_types=[
        pltpu.VMEM((CPTP, CH), jnp.int32),
        pltpu.VMEM((CH, 16), jnp.float32),
        pltpu.VMEM_SHARED((NPAD, 16), jnp.float32),
        pltpu.SemaphoreType.DMA,
    ],
    compiler_params=_SC_PARAMS,
)
def _deg(dst, ones, zeros, out, idx_d, obuf, acc, ssem):
    c = lax.axis_index("c")
    s = lax.axis_index("s")
    wid = c * NS + s
    r0 = s * RPT
    pltpu.sync_copy(zeros, acc.at[pl.ds(r0, RPT)])
    pltpu.sync_copy(dst.at[wid], idx_d)
    pltpu.sync_copy(ones, obuf)
    plsc.subcore_barrier()

    for j in (0, 1):
        pltpu.async_copy(obuf, acc.at[idx_d.at[j]], ssem, add=True)

    def slot(j, carry):
        pltpu.async_copy(obuf, acc.at[idx_d.at[j]], ssem, add=True)
        pltpu.make_async_copy(obuf, acc.at[idx_d.at[j - 2]], ssem).wait()
        return carry

    lax.fori_loop(2, CPT, slot, 0)
    pltpu.make_async_copy(obuf, acc.at[idx_d.at[CPT - 2]], ssem).wait()
    pltpu.make_async_copy(obuf, acc.at[idx_d.at[CPT - 1]], ssem).wait()
    plsc.subcore_barrier()
    pltpu.sync_copy(acc.at[pl.ds(r0, RPT)], out.at[c, pl.ds(r0, RPT)])


def _proj_body(x, w, out):
    out[...] = jnp.dot(x[...], w[...], preferred_element_type=jnp.float32)


_proj = pl.pallas_call(
    _proj_body, out_shape=jax.ShapeDtypeStruct((N, 64), jnp.float32)
)


def _scale1_body(p, h, h_out, dinv_out):
    deg = p[0, :, 0:1] + p[1, :, 0:1] + 1.0
    dinv = lax.rsqrt(deg)
    dinv_out[...] = dinv
    h_out[0:N, :] = h[...] * dinv[0:N]
    h_out[N:NPAD, :] = jnp.zeros((NPAD - N, 64), jnp.float32)


_scale1 = pl.pallas_call(
    _scale1_body,
    out_shape=(
        jax.ShapeDtypeStruct((NPAD, 64), jnp.float32),
        jax.ShapeDtypeStruct((NPAD, 1), jnp.float32),
    ),
)


def _mid_body(p, hp, dinv, b, g, be, w, out):
    t = dinv[0:N] * (p[0, 0:N] + p[1, 0:N] + hp[0:N]) + b[...]
    mu = jnp.mean(t, axis=0, keepdims=True)
    var = jnp.mean((t - mu) ** 2, axis=0, keepdims=True)
    t = g[...] * (t - mu) * lax.rsqrt(var + EPS) + be[...]
    t = jnp.maximum(t, 0.0)
    h = jnp.dot(t, w[...], preferred_element_type=jnp.float32)
    out[0:N, :] = h * dinv[0:N]
    out[N:NPAD, :] = jnp.zeros((NPAD - N, h.shape[1]), jnp.float32)


def _make_mid(wout):
    return pl.pallas_call(
        _mid_body,
        out_shape=jax.ShapeDtypeStruct((NPAD, wout), jnp.float32),
    )


_mid64 = _make_mid(64)
_mid16 = _make_mid(16)


def _final_body(p, hp, dinv, b, g, be, out):
    t = dinv[0:N] * (p[0, 0:N] + p[1, 0:N] + hp[0:N])
    t = t[:, 0:2] + b[...]
    mu = jnp.mean(t, axis=0, keepdims=True)
    var = jnp.mean((t - mu) ** 2, axis=0, keepdims=True)
    out[...] = g[...] * (t - mu) * lax.rsqrt(var + EPS) + be[...]


_final = pl.pallas_call(
    _final_body,
    out_shape=jax.ShapeDtypeStruct((N, 2), jnp.float32),
)


def kernel(x, edge_index, W1, b1, g1, be1, W2, b2, g2, be2, W3, b3, g3, be3):
    src = edge_index[0].astype(jnp.int32)
    dst = edge_index[1].astype(jnp.int32)
    def _lay(e):
        fpad = N + jnp.arange(TILES * CPT * CH - E, dtype=jnp.int32) % (NPAD - N)
        main = jnp.concatenate([e, fpad]).reshape(TILES, CPT, CH)
        tail = (N + jnp.arange(TILES * 2 * CH, dtype=jnp.int32)
                % (NPAD - N)).reshape(TILES, 2, CH)
        return jnp.concatenate([main, tail], axis=1)

    srcp = _lay(src)
    dstp = _lay(dst)

    ones = jnp.ones((CH, 16), jnp.float32)
    z16 = jnp.zeros((RPT, 16), jnp.float32)
    z64 = jnp.zeros((RPT, 64), jnp.float32)

    pdeg = _deg(dstp, ones, z16)
    h1 = _proj(x, W1)
    h1p, dinv = _scale1(pdeg, h1)

    p1 = _agg64(h1p, srcp, dstp, z64)
    h2p = _mid64(p1, h1p, dinv,
                 b1.reshape(1, -1), g1.reshape(1, -1), be1.reshape(1, -1), W2)

    p2 = _agg64(h2p, srcp, dstp, z64)
    W3p = jnp.pad(W3, ((0, 0), (0, 16 - W3.shape[1])))
    h3p = _mid16(p2, h2p, dinv,
                 b2.reshape(1, -1), g2.reshape(1, -1), be2.reshape(1, -1), W3p)

    p3 = _agg16(h3p, srcp, dstp, z16)
    out = _final(p3, h3p, dinv,
                 b3.reshape(1, -1), g3.reshape(1, -1), be3.reshape(1, -1))
    return out


# speedup vs baseline: 37.1997x; 1.0405x over previous
import functools

import jax
import jax.numpy as jnp
from jax import lax
from jax.experimental import pallas as pl
from jax.experimental.pallas import tpu as pltpu
from jax.experimental.pallas import tpu_sc as plsc

N = 10000
E = 320000
NC = 2
NS = 16
TILES = NC * NS
CH = 128
CPT = 80
NBUF = 8
PF = NBUF - 2
NPAD = 10112
RPT = NPAD // NS
EPS = 1e-5

_MESH = plsc.VectorSubcoreMesh(core_axis_name="c", subcore_axis_name="s")
_SC_PARAMS = pltpu.CompilerParams(use_tc_tiling_on_sc=False)


def _make_agg(width):
    @functools.partial(
        pl.kernel,
        out_type=jax.ShapeDtypeStruct((NC, NPAD, width), jnp.float32),
        mesh=_MESH,
        scratch_types=[
            pltpu.VMEM((CPT + PF, CH), jnp.int32),
            pltpu.VMEM((CPT, CH), jnp.int32),
            pltpu.VMEM((NBUF, CH, width), jnp.float32),
            pltpu.VMEM_SHARED((NPAD, width), jnp.float32),
            pltpu.SemaphoreType.DMA,
            pltpu.SemaphoreType.DMA,
        ],
        compiler_params=_SC_PARAMS,
    )
    def agg(table, src, dst, safe, zeros, out, idx_s, idx_d, buf, acc, gsem, ssem):
        c = lax.axis_index("c")
        s = lax.axis_index("s")
        wid = c * NS + s
        r0 = s * RPT
        pltpu.sync_copy(zeros, acc.at[pl.ds(r0, RPT)])
        pltpu.sync_copy(src.at[wid], idx_s.at[pl.ds(0, CPT)])
        pltpu.sync_copy(safe, idx_s.at[pl.ds(CPT, PF)])
        pltpu.sync_copy(dst.at[wid], idx_d)
        plsc.subcore_barrier()

        for j in range(PF):
            pltpu.async_copy(table.at[idx_s.at[j]], buf.at[j], gsem)
        for j in (0, 1):
            pltpu.make_async_copy(table.at[idx_s.at[j]], buf.at[j], gsem).wait()
            pltpu.async_copy(buf.at[j], acc.at[idx_d.at[j]], ssem, add=True)
            pltpu.async_copy(table.at[idx_s.at[j + PF]], buf.at[j + PF], gsem)

        def slot(j, carry):
            b = jnp.bitwise_and(j, NBUF - 1)
            bm2 = jnp.bitwise_and(j + PF, NBUF - 1)
            pltpu.make_async_copy(table.at[idx_s.at[j]], buf.at[b], gsem).wait()
            pltpu.async_copy(buf.at[b], acc.at[idx_d.at[j]], ssem, add=True)
            pltpu.make_async_copy(buf.at[bm2], acc.at[idx_d.at[j - 2]], ssem).wait()
            pltpu.async_copy(table.at[idx_s.at[j + PF]], buf.at[bm2], gsem)
            return carry

        lax.fori_loop(2, CPT, slot, 0)
        pltpu.make_async_copy(
            buf.at[(CPT - 2) % NBUF], acc.at[idx_d.at[CPT - 2]], ssem).wait()
        pltpu.make_async_copy(
            buf.at[(CPT - 1) % NBUF], acc.at[idx_d.at[CPT - 1]], ssem).wait()
        for k in range(PF):
            pltpu.make_async_copy(
                table.at[idx_s.at[CPT + k]], buf.at[(CPT + k) % NBUF], gsem
            ).wait()
        plsc.subcore_barrier()
        pltpu.sync_copy(acc.at[pl.ds(r0, RPT)], out.at[c, pl.ds(r0, RPT)])

    return agg


_agg64 = _make_agg(64)
_agg16 = _make_agg(16)


@functools.partial(
    pl.kernel,
    out_type=jax.ShapeDtypeStruct((NC, NPAD, 16), jnp.float32),
    mesh=_MESH,
    scratch_types=[
        pltpu.VMEM((CPT, CH), jnp.int32),
        pltpu.VMEM((CH, 16), jnp.float32),
        pltpu.VMEM_SHARED((NPAD, 16), jnp.float32),
        pltpu.SemaphoreType.DMA,
    ],
    compiler_params=_SC_PARAMS,
)
def _deg(dst, ones, zeros, out, idx_d, obuf, acc, ssem):
    c = lax.axis_index("c")
    s = lax.axis_index("s")
    wid = c * NS + s
    r0 = s * RPT
    pltpu.sync_copy(zeros, acc.at[pl.ds(r0, RPT)])
    pltpu.sync_copy(dst.at[wid], idx_d)
    pltpu.sync_copy(ones, obuf)
    plsc.subcore_barrier()

    for j in (0, 1):
        pltpu.async_copy(obuf, acc.at[idx_d.at[j]], ssem, add=True)

    def slot(j, carry):
        pltpu.async_copy(obuf, acc.at[idx_d.at[j]], ssem, add=True)
        pltpu.make_async_copy(obuf, acc.at[idx_d.at[j - 2]], ssem).wait()
        return carry

    lax.fori_loop(2, CPT, slot, 0)
    pltpu.make_async_copy(obuf, acc.at[idx_d.at[CPT - 2]], ssem).wait()
    pltpu.make_async_copy(obuf, acc.at[idx_d.at[CPT - 1]], ssem).wait()
    plsc.subcore_barrier()
    pltpu.sync_copy(acc.at[pl.ds(r0, RPT)], out.at[c, pl.ds(r0, RPT)])


def _proj_body(x, w, out):
    out[...] = jnp.dot(x[...], w[...], preferred_element_type=jnp.float32)


_proj = pl.pallas_call(
    _proj_body, out_shape=jax.ShapeDtypeStruct((N, 64), jnp.float32)
)


def _scale1_body(p, h, h_out, dinv_out):
    deg = p[0, :, 0:1] + p[1, :, 0:1] + 1.0
    dinv = lax.rsqrt(deg)
    dinv_out[...] = dinv
    h_out[0:N, :] = h[...] * dinv[0:N]
    h_out[N:NPAD, :] = jnp.zeros((NPAD - N, 64), jnp.float32)


_scale1 = pl.pallas_call(
    _scale1_body,
    out_shape=(
        jax.ShapeDtypeStruct((NPAD, 64), jnp.float32),
        jax.ShapeDtypeStruct((NPAD, 1), jnp.float32),
    ),
)


def _mid_body(p, hp, dinv, b, g, be, w, out):
    t = dinv[0:N] * (p[0, 0:N] + p[1, 0:N] + hp[0:N]) + b[...]
    mu = jnp.mean(t, axis=0, keepdims=True)
    var = jnp.mean((t - mu) ** 2, axis=0, keepdims=True)
    t = g[...] * (t - mu) * lax.rsqrt(var + EPS) + be[...]
    t = jnp.maximum(t, 0.0)
    h = jnp.dot(t, w[...], preferred_element_type=jnp.float32)
    out[0:N, :] = h * dinv[0:N]
    out[N:NPAD, :] = jnp.zeros((NPAD - N, h.shape[1]), jnp.float32)


def _make_mid(wout):
    return pl.pallas_call(
        _mid_body,
        out_shape=jax.ShapeDtypeStruct((NPAD, wout), jnp.float32),
    )


_mid64 = _make_mid(64)
_mid16 = _make_mid(16)


def _final_body(p, hp, dinv, b, g, be, out):
    t = dinv[0:N] * (p[0, 0:N] + p[1, 0:N] + hp[0:N])
    t = t[:, 0:2] + b[...]
    mu = jnp.mean(t, axis=0, keepdims=True)
    var = jnp.mean((t - mu) ** 2, axis=0, keepdims=True)
    out[...] = g[...] * (t - mu) * lax.rsqrt(var + EPS) + be[...]


_final = pl.pallas_call(
    _final_body,
    out_shape=jax.ShapeDtypeStruct((N, 2), jnp.float32),
)


def kernel(x, edge_index, W1, b1, g1, be1, W2, b2, g2, be2, W3, b3, g3, be3):
    src = edge_index[0].astype(jnp.int32)
    dst = edge_index[1].astype(jnp.int32)
    fpad = N + jnp.arange(TILES * CPT * CH - E, dtype=jnp.int32) % (NPAD - N)
    srcp = jnp.concatenate([src, fpad]).reshape(TILES, CPT, CH)
    dstp = jnp.concatenate([dst, fpad]).reshape(TILES, CPT, CH)
    safe = (N + jnp.arange(PF * CH, dtype=jnp.int32)
            % (NPAD - N)).reshape(PF, CH)

    ones = jnp.ones((CH, 16), jnp.float32)
    z16 = jnp.zeros((RPT, 16), jnp.float32)
    z64 = jnp.zeros((RPT, 64), jnp.float32)

    pdeg = _deg(dstp, ones, z16)
    h1 = _proj(x, W1)
    h1p, dinv = _scale1(pdeg, h1)

    p1 = _agg64(h1p, srcp, dstp, safe, z64)
    h2p = _mid64(p1, h1p, dinv,
                 b1.reshape(1, -1), g1.reshape(1, -1), be1.reshape(1, -1), W2)

    p2 = _agg64(h2p, srcp, dstp, safe, z64)
    W3p = jnp.pad(W3, ((0, 0), (0, 16 - W3.shape[1])))
    h3p = _mid16(p2, h2p, dinv,
                 b2.reshape(1, -1), g2.reshape(1, -1), be2.reshape(1, -1), W3p)

    p3 = _agg16(h3p, srcp, dstp, safe, z16)
    out = _final(p3, h3p, dinv,
                 b3.reshape(1, -1), g3.reshape(1, -1), be3.reshape(1, -1))
    return out


# speedup vs baseline: 38.2252x; 1.0276x over previous
import functools

import jax
import jax.numpy as jnp
from jax import lax
from jax.experimental import pallas as pl
from jax.experimental.pallas import tpu as pltpu
from jax.experimental.pallas import tpu_sc as plsc

N = 10000
E = 320000
NC = 2
NS = 16
TILES = NC * NS
CH = 125
CPT = 80
PFMAX = 6
NPAD = 10112
RPT = NPAD // NS
EPS = 1e-5

_MESH = plsc.VectorSubcoreMesh(core_axis_name="c", subcore_axis_name="s")
_SC_PARAMS = pltpu.CompilerParams(use_tc_tiling_on_sc=False)


def _make_agg(width, nbuf):
    pf = nbuf - 2

    @functools.partial(
        pl.kernel,
        out_type=jax.ShapeDtypeStruct((NC, NPAD, width), jnp.float32),
        mesh=_MESH,
        scratch_types=[
            pltpu.VMEM((CPT + pf, CH), jnp.int32),
            pltpu.VMEM((CPT, CH), jnp.int32),
            pltpu.VMEM((nbuf, CH, width), jnp.float32),
            pltpu.VMEM_SHARED((NPAD, width), jnp.float32),
            pltpu.SemaphoreType.DMA,
            pltpu.SemaphoreType.DMA,
        ],
        compiler_params=_SC_PARAMS,
    )
    def agg(table, src, dst, safe, zeros, out, idx_s, idx_d, buf, acc, gsem, ssem):
        c = lax.axis_index("c")
        s = lax.axis_index("s")
        wid = c * NS + s
        r0 = s * RPT
        pltpu.sync_copy(zeros, acc.at[pl.ds(r0, RPT)])
        pltpu.sync_copy(src.at[wid], idx_s.at[pl.ds(0, CPT)])
        pltpu.sync_copy(safe.at[pl.ds(0, pf)], idx_s.at[pl.ds(CPT, pf)])
        pltpu.sync_copy(dst.at[wid], idx_d)
        plsc.subcore_barrier()

        for j in range(pf):
            pltpu.async_copy(table.at[idx_s.at[j]], buf.at[j], gsem)
        for j in (0, 1):
            pltpu.make_async_copy(table.at[idx_s.at[j]], buf.at[j], gsem).wait()
            pltpu.async_copy(buf.at[j], acc.at[idx_d.at[j]], ssem, add=True)
            pltpu.async_copy(table.at[idx_s.at[j + pf]],
                             buf.at[(j + pf) % nbuf], gsem)

        def slot(j, carry):
            b = jnp.bitwise_and(j, nbuf - 1)
            bm2 = jnp.bitwise_and(j + pf, nbuf - 1)
            pltpu.make_async_copy(table.at[idx_s.at[j]], buf.at[b], gsem).wait()
            pltpu.async_copy(buf.at[b], acc.at[idx_d.at[j]], ssem, add=True)
            pltpu.make_async_copy(buf.at[bm2], acc.at[idx_d.at[j - 2]], ssem).wait()
            pltpu.async_copy(table.at[idx_s.at[j + pf]], buf.at[bm2], gsem)
            return carry

        lax.fori_loop(2, CPT, slot, 0)
        pltpu.make_async_copy(
            buf.at[(CPT - 2) % nbuf], acc.at[idx_d.at[CPT - 2]], ssem).wait()
        pltpu.make_async_copy(
            buf.at[(CPT - 1) % nbuf], acc.at[idx_d.at[CPT - 1]], ssem).wait()
        for k in range(pf):
            pltpu.make_async_copy(
                table.at[idx_s.at[CPT + k]], buf.at[(CPT + k) % nbuf], gsem
            ).wait()
        plsc.subcore_barrier()
        pltpu.sync_copy(acc.at[pl.ds(r0, RPT)], out.at[c, pl.ds(r0, RPT)])

    return agg


_agg64 = _make_agg(64, 4)
_agg16 = _make_agg(16, 8)


@functools.partial(
    pl.kernel,
    out_type=jax.ShapeDtypeStruct((NC, NPAD, 16), jnp.float32),
    mesh=_MESH,
    scratch_types=[
        pltpu.VMEM((CPT, CH), jnp.int32),
        pltpu.VMEM((CH, 16), jnp.float32),
        pltpu.VMEM_SHARED((NPAD, 16), jnp.float32),
        pltpu.SemaphoreType.DMA,
    ],
    compiler_params=_SC_PARAMS,
)
def _deg(dst, ones, zeros, out, idx_d, obuf, acc, ssem):
    c = lax.axis_index("c")
    s = lax.axis_index("s")
    wid = c * NS + s
    r0 = s * RPT
    pltpu.sync_copy(zeros, acc.at[pl.ds(r0, RPT)])
    pltpu.sync_copy(dst.at[wid], idx_d)
    pltpu.sync_copy(ones, obuf)
    plsc.subcore_barrier()

    for j in (0, 1):
        pltpu.async_copy(obuf, acc.at[idx_d.at[j]], ssem, add=True)

    def slot(j, carry):
        pltpu.async_copy(obuf, acc.at[idx_d.at[j]], ssem, add=True)
        pltpu.make_async_copy(obuf, acc.at[idx_d.at[j - 2]], ssem).wait()
        return carry

    lax.fori_loop(2, CPT, slot, 0)
    pltpu.make_async_copy(obuf, acc.at[idx_d.at[CPT - 2]], ssem).wait()
    pltpu.make_async_copy(obuf, acc.at[idx_d.at[CPT - 1]], ssem).wait()
    plsc.subcore_barrier()
    pltpu.sync_copy(acc.at[pl.ds(r0, RPT)], out.at[c, pl.ds(r0, RPT)])


def _proj_body(x, w, out):
    out[...] = jnp.dot(x[...], w[...], preferred_element_type=jnp.float32)


_proj = pl.pallas_call(
    _proj_body, out_shape=jax.ShapeDtypeStruct((N, 64), jnp.float32)
)


def _scale1_body(p, h, h_out, dinv_out):
    deg = p[0, :, 0:1] + p[1, :, 0:1] + 1.0
    dinv = lax.rsqrt(deg)
    dinv_out[...] = dinv
    h_out[0:N, :] = h[...] * dinv[0:N]
    h_out[N:NPAD, :] = jnp.zeros((NPAD - N, 64), jnp.float32)


_scale1 = pl.pallas_call(
    _scale1_body,
    out_shape=(
        jax.ShapeDtypeStruct((NPAD, 64), jnp.float32),
        jax.ShapeDtypeStruct((NPAD, 1), jnp.float32),
    ),
)


def _mid_body(p, hp, dinv, b, g, be, w, out):
    t = dinv[0:N] * (p[0, 0:N] + p[1, 0:N] + hp[0:N]) + b[...]
    mu = jnp.mean(t, axis=0, keepdims=True)
    var = jnp.mean((t - mu) ** 2, axis=0, keepdims=True)
    t = g[...] * (t - mu) * lax.rsqrt(var + EPS) + be[...]
    t = jnp.maximum(t, 0.0)
    h = jnp.dot(t, w[...], preferred_element_type=jnp.float32)
    out[0:N, :] = h * dinv[0:N]
    out[N:NPAD, :] = jnp.zeros((NPAD - N, h.shape[1]), jnp.float32)


def _make_mid(wout):
    return pl.pallas_call(
        _mid_body,
        out_shape=jax.ShapeDtypeStruct((NPAD, wout), jnp.float32),
    )


_mid64 = _make_mid(64)
_mid16 = _make_mid(16)


def _final_body(p, hp, dinv, b, g, be, out):
    t = dinv[0:N] * (p[0, 0:N] + p[1, 0:N] + hp[0:N])
    t = t[:, 0:2] + b[...]
    mu = jnp.mean(t, axis=0, keepdims=True)
    var = jnp.mean((t - mu) ** 2, axis=0, keepdims=True)
    out[...] = g[...] * (t - mu) * lax.rsqrt(var + EPS) + be[...]


_final = pl.pallas_call(
    _final_body,
    out_shape=jax.ShapeDtypeStruct((N, 2), jnp.float32),
)


def kernel(x, edge_index, W1, b1, g1, be1, W2, b2, g2, be2, W3, b3, g3, be3):
    srcp = edge_index[0].astype(jnp.int32).reshape(TILES, CPT, CH)
    dstp = edge_index[1].astype(jnp.int32).reshape(TILES, CPT, CH)
    safe = (N + jnp.arange(PFMAX * CH, dtype=jnp.int32)
            % (NPAD - N)).reshape(PFMAX, CH)

    ones = jnp.ones((CH, 16), jnp.float32)
    z16 = jnp.zeros((RPT, 16), jnp.float32)
    z64 = jnp.zeros((RPT, 64), jnp.float32)

    pdeg = _deg(dstp, ones, z16)
    h1 = _proj(x, W1)
    h1p, dinv = _scale1(pdeg, h1)

    p1 = _agg64(h1p, srcp, dstp, safe, z64)
    h2p = _mid64(p1, h1p, dinv,
                 b1.reshape(1, -1), g1.reshape(1, -1), be1.reshape(1, -1), W2)

    p2 = _agg64(h2p, srcp, dstp, safe, z64)
    W3p = jnp.pad(W3, ((0, 0), (0, 16 - W3.shape[1])))
    h3p = _mid16(p2, h2p, dinv,
                 b2.reshape(1, -1), g2.reshape(1, -1), be2.reshape(1, -1), W3p)

    p3 = _agg16(h3p, srcp, dstp, safe, z16)
    out = _final(p3, h3p, dinv,
                 b3.reshape(1, -1), g3.reshape(1, -1), be3.reshape(1, -1))
    return out
